# baseline (speedup 1.0000x reference)
import functools

import jax
import jax.numpy as jnp
from jax.experimental import pallas as pl
from jax.experimental.pallas import tpu as pltpu

_PACKED = True


def _body(cls_ref, cls_out, reg_out, pos_out):
    j = pl.program_id(1)
    s = jnp.reshape(jnp.sum(cls_ref[0]), (1, 1, 1))

    @pl.when(j == 0)
    def _init():
        cls_out[...] = s
        reg_out[...] = jnp.ones((1, 1, 1), jnp.float32)
        pos_out[...] = jnp.ones((1, 1, 1), jnp.float32)

    @pl.when(j != 0)
    def _acc():
        cls_out[...] += s


def kernel(cls_heads, reg_heads, batch_anchors, annotations):
    b, a, c = cls_heads.shape
    if _PACKED:
        rows = a * c // 128
        x = cls_heads.reshape(b, rows, 128)
        blk_rows = 2560
        nb = -(-rows // blk_rows)
        spec = pl.BlockSpec((1, blk_rows, 128), lambda i, j: (i, j, 0))
        grid = (b, nb)
    else:
        x = cls_heads
        blk = 4096
        nb = -(-a // blk)
        spec = pl.BlockSpec((1, blk, c), lambda i, j: (i, j, 0))
        grid = (b, nb)
    call = pl.pallas_call(
        _body,
        grid=grid,
        in_specs=[spec],
        out_specs=[
            pl.BlockSpec((1, 1, 1), lambda i, j: (i, 0, 0)),
            pl.BlockSpec((1, 1, 1), lambda i, j: (i, 0, 0)),
            pl.BlockSpec((1, 1, 1), lambda i, j: (i, 0, 0)),
        ],
        out_shape=[
            jax.ShapeDtypeStruct((b, 1, 1), jnp.float32),
            jax.ShapeDtypeStruct((b, 1, 1), jnp.float32),
            jax.ShapeDtypeStruct((b, 1, 1), jnp.float32),
        ],
        compiler_params=pltpu.CompilerParams(
            dimension_semantics=("parallel", "arbitrary")),
    )
    cs, rs, ps = call(x)
    return (jnp.mean(cs[:, 0, 0] / ps[:, 0, 0]),
            jnp.mean(rs[:, 0, 0] / ps[:, 0, 0]))


# speedup vs baseline: 2.1176x; 2.1176x over previous
import functools

import jax
import jax.numpy as jnp
from jax.experimental import pallas as pl
from jax.experimental.pallas import tpu as pltpu

_ALPHA = 0.25
_BETA = 1.0 / 9.0
_EPS = 1e-4
_BLK = 8192


def _loss_body(ann_ref, anc_ref, cls_ref, reg_ref, cls_out, reg_out, pos_out,
               *, blk, total_a, ngt):
    j = pl.program_id(1)
    tr = blk // 128

    ax0 = anc_ref[0, 0]
    ay0 = anc_ref[0, 1]
    ax1 = anc_ref[0, 2]
    ay1 = anc_ref[0, 3]
    area_a = (ax1 - ax0) * (ay1 - ay0)

    best = jnp.full((tr, 128), -1.0, dtype=jnp.float32)
    bx0 = jnp.zeros((tr, 128), dtype=jnp.float32)
    by0 = jnp.zeros((tr, 128), dtype=jnp.float32)
    bx1 = jnp.ones((tr, 128), dtype=jnp.float32)
    by1 = jnp.ones((tr, 128), dtype=jnp.float32)
    bcl = jnp.zeros((tr, 128), dtype=jnp.float32)
    for n in range(ngt):
        g0 = ann_ref[0, 0, 5 * n + 0]
        g1 = ann_ref[0, 0, 5 * n + 1]
        g2 = ann_ref[0, 0, 5 * n + 2]
        g3 = ann_ref[0, 0, 5 * n + 3]
        gc = ann_ref[0, 0, 5 * n + 4]
        area_g = (g2 - g0) * (g3 - g1)
        ow = jnp.maximum(jnp.minimum(ax1, g2) - jnp.maximum(ax0, g0), 0.0)
        oh = jnp.maximum(jnp.minimum(ay1, g3) - jnp.maximum(ay0, g1), 0.0)
        inter = ow * oh
        union = jnp.maximum(area_a + area_g - inter, 1e-4)
        iou = inter / union
        upd = iou > best
        best = jnp.where(upd, iou, best)
        bx0 = jnp.where(upd, g0, bx0)
        by0 = jnp.where(upd, g1, by0)
        bx1 = jnp.where(upd, g2, bx1)
        by1 = jnp.where(upd, g3, by1)
        bcl = jnp.where(upd, gc, bcl)

    assign = jnp.where(best < 0.4, 0.0, -1.0)
    assign = jnp.where(best >= 0.5, bcl + 1.0, assign)

    aidx = (jax.lax.broadcasted_iota(jnp.int32, (tr, 128), 0) * 128
            + jax.lax.broadcasted_iota(jnp.int32, (tr, 128), 1))
    in_rng = (j * blk + aidx) < total_a
    valid = in_rng & (assign >= 0.0)
    pos = in_rng & (assign > 0.0)
    posf = pos.astype(jnp.float32)

    aw = ax1 - ax0
    ah = ay1 - ay0
    gw = bx1 - bx0
    gh = by1 - by0
    tx = ((bx0 + 0.5 * gw) - (ax0 + 0.5 * aw)) / aw / 0.1
    ty = ((by0 + 0.5 * gh) - (ay0 + 0.5 * ah)) / ah / 0.1
    tw = jnp.log(gw / aw) / 0.2
    th = jnp.log(gh / ah) / 0.2

    def sl1(d):
        return jnp.where(d >= _BETA, d - 0.5 * _BETA, 0.5 * d * d / _BETA)

    per_anchor = (sl1(jnp.abs(reg_ref[0, 0] - tx))
                  + sl1(jnp.abs(reg_ref[0, 1] - ty))
                  + sl1(jnp.abs(reg_ref[0, 2] - tw))
                  + sl1(jnp.abs(reg_ref[0, 3] - th))) * 0.25
    reg_partial = jnp.sum(jnp.where(pos, per_anchor, 0.0))
    pos_partial = jnp.sum(posf)

    x = cls_ref[0]
    c = x.shape[1]
    p = jnp.clip(x, _EPS, 1.0 - _EPS).reshape(tr, 128, c)
    row_tile = jnp.sum(p * p * jnp.log2(1.0 - p), axis=2)

    t = assign.astype(jnp.int32) - 1
    cio = jax.lax.broadcasted_iota(jnp.int32, (tr, 128, c), 2)
    p_t = jnp.sum(jnp.where(t[:, :, None] == cio, p, 0.0), axis=2)

    p_t = jnp.where(pos, p_t, 0.5)
    neg_t = (1.0 - _ALPHA) * p_t * p_t * (-jnp.log(1.0 - p_t))
    pos_t = _ALPHA * (1.0 - p_t) * (1.0 - p_t) * (-jnp.log(p_t))
    corr = jnp.where(pos, pos_t - neg_t, 0.0)

    _LN2 = 0.6931471805599453
    cls_partial = ((_ALPHA - 1.0) * _LN2
                   * jnp.sum(jnp.where(valid, row_tile, 0.0))
                   + jnp.sum(corr))

    cls_out[...] = jnp.reshape(cls_partial, (1, 1, 1, 1))
    reg_out[...] = jnp.reshape(reg_partial, (1, 1, 1, 1))
    pos_out[...] = jnp.reshape(pos_partial, (1, 1, 1, 1))


def _build_call(b, a, c, n, blk, a2, interpret=False):
    tr = blk // 128
    body = functools.partial(_loss_body, blk=blk, total_a=a, ngt=n)
    return pl.pallas_call(
        body,
        grid=(b, a2 // blk),
        in_specs=[
            pl.BlockSpec((1, 1, n * 5), lambda i, j: (i, 0, 0),
                         memory_space=pltpu.SMEM),
            pl.BlockSpec((1, 4, tr, 128), lambda i, j: (i, 0, j, 0)),
            pl.BlockSpec((1, blk, c), lambda i, j: (i, j, 0)),
            pl.BlockSpec((1, 4, tr, 128), lambda i, j: (i, 0, j, 0)),
        ],
        out_specs=[
            pl.BlockSpec((1, 1, 1, 1), lambda i, j: (i, j, 0, 0)),
            pl.BlockSpec((1, 1, 1, 1), lambda i, j: (i, j, 0, 0)),
            pl.BlockSpec((1, 1, 1, 1), lambda i, j: (i, j, 0, 0)),
        ],
        out_shape=[
            jax.ShapeDtypeStruct((b, a2 // blk, 1, 1), jnp.float32),
            jax.ShapeDtypeStruct((b, a2 // blk, 1, 1), jnp.float32),
            jax.ShapeDtypeStruct((b, a2 // blk, 1, 1), jnp.float32),
        ],
        interpret=interpret,
        compiler_params=pltpu.CompilerParams(
            dimension_semantics=("parallel", "parallel")),
    )


def _prep(batch_anchors, reg_heads, a2):
    b, a, _ = batch_anchors.shape
    pad = a2 - a
    anc_t = batch_anchors.transpose(0, 2, 1)
    reg_t = reg_heads.transpose(0, 2, 1)
    if pad:
        pad_box = jnp.broadcast_to(
            jnp.array([0.0, 0.0, 128.0, 128.0], jnp.float32)[None, :, None],
            (b, 4, pad))
        anc_t = jnp.concatenate([anc_t, pad_box], axis=2)
        reg_t = jnp.concatenate(
            [reg_t, jnp.zeros((b, 4, pad), jnp.float32)], axis=2)
    return (anc_t.reshape(b, 4, a2 // 128, 128),
            reg_t.reshape(b, 4, a2 // 128, 128))


def kernel(cls_heads, reg_heads, batch_anchors, annotations):
    b, a, c = cls_heads.shape
    n = annotations.shape[1]
    blk = _BLK
    a2 = -(-a // blk) * blk
    anc_t, reg_t = _prep(batch_anchors, reg_heads, a2)
    ann_s = annotations.reshape(b, 1, n * 5)
    call = _build_call(b, a, c, n, blk, a2)
    cls_sums, reg_sums, pos_sums = call(ann_s, anc_t, cls_heads, reg_t)
    cls_b = jnp.sum(cls_sums[:, :, 0, 0], axis=1)
    reg_b = jnp.sum(reg_sums[:, :, 0, 0], axis=1)
    pos_b = jnp.sum(pos_sums[:, :, 0, 0], axis=1)
    return (jnp.mean(cls_b / pos_b), jnp.mean(reg_b / pos_b))
